# baseline (speedup 1.0000x reference)
import functools

import jax
import jax.numpy as jnp
from jax.experimental import pallas as pl
from jax.experimental.pallas import tpu as pltpu

_BETA = 1.0 / 9.0


def _smooth_l1(diff):
    ad = jnp.abs(diff)
    t = jnp.minimum(ad, _BETA)
    return ad - t + t * t * (0.5 / _BETA)


def _loss_body(cls_ref, b2d_ref, b3d_ref, b2dt_ref, b3dt_ref, w_ref, lab_ref,
               out_ref, acc_ref):
    pid = pl.program_id(0)
    nsteps = pl.num_programs(0)

    @pl.when(pid == 0)
    def _init():
        acc_ref[0] = 0.0
        acc_ref[1] = 0.0
        acc_ref[2] = 0.0
        acc_ref[3] = 0.0
        acc_ref[4] = 0.0

    lab = lab_ref[...]
    x = cls_ref[...]
    m = jnp.max(x, axis=1, keepdims=True)
    e = jnp.exp(x - m)
    lse = jnp.log(jnp.sum(e, axis=1, keepdims=True)) + m
    onehot = (jax.lax.broadcasted_iota(jnp.int32, x.shape, 1) == lab)
    sel = jnp.sum(jnp.where(onehot, x, 0.0), axis=1, keepdims=True)
    ce = lse - sel

    labf = lab
    active = (labf >= 0).astype(jnp.float32)
    fg = (labf > 0).astype(jnp.float32)
    w = fg * w_ref[...]

    l2d = jnp.sum(_smooth_l1(b2d_ref[...] - b2dt_ref[...]), axis=1,
                  keepdims=True)
    l3d = jnp.sum(_smooth_l1(b3d_ref[...] - b3dt_ref[...]), axis=1,
                  keepdims=True)

    acc_ref[0] += jnp.sum(ce * active)
    acc_ref[1] += jnp.sum(active)
    acc_ref[2] += jnp.sum(fg)
    acc_ref[3] += jnp.sum(l2d * w)
    acc_ref[4] += jnp.sum(l3d * w)

    @pl.when(pid == nsteps - 1)
    def _fini():
        cls_loss = acc_ref[0] / jnp.maximum(acc_ref[1], 1.0)
        nfg = jnp.maximum(acc_ref[2], 1.0)
        out_ref[0, 0] = cls_loss + (acc_ref[3] + acc_ref[4]) / nfg


@functools.partial(jax.jit, static_argnames=())
def kernel(cls, bbox_2d, bbox_3d, bbox_2d_tar, bbox_3d_tar, bbox_weights,
           labels):
    B, N, C = cls.shape
    A = B * N
    cls2 = cls.reshape(A, C)
    b2d = bbox_2d.reshape(A, 4)
    b3d = bbox_3d.reshape(A, 11)
    b2dt = bbox_2d_tar.reshape(A, 4)
    b3dt = bbox_3d_tar.reshape(A, 11)
    w2 = bbox_weights.reshape(A, 1)
    lab2 = labels.reshape(A, 1).astype(jnp.int32)

    BLK = 2560
    assert A % BLK == 0
    grid = (A // BLK,)

    def bs(c):
        return pl.BlockSpec((BLK, c), lambda i: (i, 0))

    out = pl.pallas_call(
        _loss_body,
        grid=grid,
        in_specs=[bs(C), bs(4), bs(11), bs(4), bs(11), bs(1), bs(1)],
        out_specs=pl.BlockSpec(memory_space=pltpu.SMEM),
        out_shape=jax.ShapeDtypeStruct((1, 1), jnp.float32),
        scratch_shapes=[pltpu.SMEM((8,), jnp.float32)],
        compiler_params=pltpu.CompilerParams(
            dimension_semantics=("arbitrary",)),
    )(cls2, b2d, b3d, b2dt, b3dt, w2, lab2)
    return out.reshape(())
